# baseline (speedup 1.0000x reference)
import jax
import jax.numpy as jnp
from jax import lax
from jax.experimental import pallas as pl
from jax.experimental.pallas import tpu as pltpu
from jax.experimental.pallas import tpu_sc as plsc

BATCH = 4096
FIELDS = 26
DIM = 16
DP1 = DIM + 1
NW = 32
BPW = BATCH // NW
NG = BPW // 16
VOCAB = 1000000
LPB = 8192
RPB = 7 * LPB
NBLK = -(-VOCAB // RPB)
PACKED_LINES = NBLK * LPB
GROWS = PACKED_LINES * 8


def _relayout_body(t_ref, out_ref):
    x = t_ref[...]
    y = jnp.concatenate(
        [lax.slice(x, (0, LPB * k), (DP1, LPB * (k + 1))) for k in range(7)],
        axis=0)
    yt = jnp.transpose(y, (1, 0))
    out_ref[:, pl.ds(0, 7 * DP1)] = yt


def _relayout(table_t):
    return pl.pallas_call(
        _relayout_body,
        grid=(NBLK,),
        in_specs=[pl.BlockSpec((DP1, RPB), lambda i: (0, i))],
        out_specs=pl.BlockSpec((LPB, 128), lambda i: (i, 0)),
        out_shape=jax.ShapeDtypeStruct((PACKED_LINES, 128), jnp.float32),
    )(table_t)


def _split_rpk(rv):
    q = rv >> 13
    i_b = (q * 9363) >> 16
    k_b = q - i_b * 7
    p_b = rv & 8191
    return i_b, k_b, p_b


def _fm_body(g16_hbm, idx_hbm, out_hbm,
             idx_v, g0f, g1f, win_all, out_v, sem):
    wid = lax.axis_index("s") * 2 + lax.axis_index("c")
    zeros16 = jnp.zeros((16,), jnp.float32)
    iota = lax.iota(jnp.int32, 16)
    zerov = jnp.zeros((16,), jnp.int32)

    pltpu.sync_copy(idx_hbm.at[pl.ds(wid * BPW, BPW)], idx_v)

    def stage_fire(j, c):
        jv = zerov + j
        for g in range(NG):
            rv = plsc.load_gather(idx_v, [g * 16 + iota, jv])
            i_b, k_b, p_b = _split_rpk(rv)
            g0 = (i_b << 16) + (p_b << 3) + k_b
            g0f[pl.ds(j * BPW + g * 16, 16)] = g0
            g1f[pl.ds(j * BPW + g * 16, 16)] = g0 + 1
        pltpu.async_copy(g16_hbm.at[g0f.at[pl.ds(j * BPW, BPW)]],
                         win_all.at[j, 0], sem)
        pltpu.async_copy(g16_hbm.at[g1f.at[pl.ds(j * BPW, BPW)]],
                         win_all.at[j, 1], sem)
        return c

    lax.fori_loop(0, FIELDS, stage_fire, 0)

    def drain(j, c):
        pltpu.make_async_copy(g16_hbm.at[g0f.at[pl.ds(j * BPW, BPW)]],
                              win_all.at[j, 0], sem).wait()
        pltpu.make_async_copy(g16_hbm.at[g1f.at[pl.ds(j * BPW, BPW)]],
                              win_all.at[j, 1], sem).wait()
        return c

    lax.fori_loop(0, FIELDS, drain, 0)

    def bg_body(bg, c):
        bvec = bg * 16 + iota

        def jbody(j, carry):
            jv = zerov + j
            rv = plsc.load_gather(idx_v, [bvec, jv])
            _, k_b, _ = _split_rpk(rv)
            vs = []
            out = []
            for d in range(DP1):
                d0 = k_b + d
                half = d0 >> 4
                pos = d0 & 15
                v = plsc.load_gather(win_all, [jv, half, bvec, pos])
                vs.append(v)
                out.append(carry[d] + v)
            for d in range(DIM):
                out.append(carry[DP1 + d] + vs[d] * vs[d])
            return tuple(out)

        init = tuple(zeros16 for _ in range(DP1 + DIM))
        acc = lax.fori_loop(0, FIELDS, jbody, init)

        plsc.store_scatter(out_v, [bvec, zerov], acc[DIM])
        for d in range(DIM):
            s = acc[d]
            val = 0.5 * (s * s - acc[DP1 + d])
            plsc.store_scatter(out_v, [bvec, zerov + (d + 1)], val)
        return c

    lax.fori_loop(0, NG, bg_body, 0)

    pltpu.sync_copy(out_v, out_hbm.at[pl.ds(wid * BPW, BPW)])


def kernel(inputs, kernel):
    packed = _relayout(kernel.T)
    g16 = packed.reshape(GROWS, 16)
    mesh = plsc.VectorSubcoreMesh(core_axis_name="c", subcore_axis_name="s")
    return pl.kernel(
        _fm_body,
        mesh=mesh,
        compiler_params=pltpu.CompilerParams(
            needs_layout_passes=False, use_tc_tiling_on_sc=False
        ),
        out_type=jax.ShapeDtypeStruct((BATCH, DP1), jnp.float32),
        scratch_types=[
            pltpu.VMEM((BPW, FIELDS), jnp.int32),
            pltpu.VMEM((FIELDS * BPW,), jnp.int32),
            pltpu.VMEM((FIELDS * BPW,), jnp.int32),
            pltpu.VMEM((FIELDS, 2, BPW, 16), jnp.float32),
            pltpu.VMEM((BPW, DP1), jnp.float32),
            pltpu.SemaphoreType.DMA,
        ],
    )(g16, inputs)
